# baseline (speedup 1.0000x reference)
import functools

import jax
import jax.numpy as jnp
from jax.experimental import pallas as pl
from jax.experimental.pallas import tpu as pltpu
from jax.experimental.pallas import tpu_sc as plsc

VOCAB_SIZE = 1000000
EMB_DIM = 64
CTX = 200
HID = 128

V_TILE = 20000
N_TILES = VOCAB_SIZE // V_TILE

IDX_PAD = 256


_SC_CORES = 2
_SC_SUBCORES = 16
_SC_WORKERS = _SC_CORES * _SC_SUBCORES
_B_PER_W = IDX_PAD // _SC_WORKERS


def _sc_gather(emb_table, idx):
    mesh = plsc.VectorSubcoreMesh(core_axis_name="c", subcore_axis_name="s")

    @functools.partial(
        pl.kernel,
        out_type=jax.ShapeDtypeStruct((IDX_PAD, 2 * EMB_DIM), jnp.float32),
        mesh=mesh,
        scratch_types=[
            pltpu.VMEM((_B_PER_W,), jnp.int32),
            pltpu.VMEM((_B_PER_W, 2 * EMB_DIM), jnp.float32),
            pltpu.SemaphoreType.DMA,
        ],
    )
    def gather_kernel(emb_hbm, idx_hbm, out_hbm, idx_v, rows_v, sem):
        wid = jax.lax.axis_index("s") * _SC_CORES + jax.lax.axis_index("c")
        base = wid * _B_PER_W
        pltpu.sync_copy(idx_hbm.at[pl.ds(base, _B_PER_W)], idx_v)
        pltpu.async_copy(emb_hbm.at[idx_v], rows_v, sem).wait()
        pltpu.sync_copy(rows_v, out_hbm.at[pl.ds(base, _B_PER_W)])

    return gather_kernel(emb_table, idx)


def _pass_a_body(e_ref, mask_ref, w1_ref, b1_ref, w2_ref, b2_ref,
                 out_ref, m_ref, s_ref, h_ref):
    i = pl.program_id(0)

    @pl.when(i == 0)
    def _():
        eb = (e_ref[...] * mask_ref[...]).astype(jnp.bfloat16)
        hpre = jax.lax.dot_general(
            eb, w1_ref[...], (((1,), (1,)), ((), ())),
            preferred_element_type=jnp.float32)
        h = jnp.maximum(hpre + b1_ref[...], 0.0)
        h_ref[...] = h.astype(jnp.bfloat16)

    w2b = w2_ref[...].astype(jnp.bfloat16)
    logits = jax.lax.dot_general(
        h_ref[...], w2b, (((1,), (1,)), ((), ())),
        preferred_element_type=jnp.float32)
    row = logits + b2_ref[0]
    out_ref[0] = row
    m = jnp.max(row)
    m_ref[...] = jnp.full((1, 1, 1), m, jnp.float32)
    s_ref[...] = jnp.full((1, 1, 1), jnp.sum(jnp.exp(row - m)), jnp.float32)


def _pass_b_body(l_ref, m_ref, s_ref, o_ref):
    m_all = m_ref[...]
    big = jnp.max(m_all)
    total = jnp.sum(s_ref[...] * jnp.exp(m_all - big))
    logz = big + jnp.log(total)
    o_ref[0] = l_ref[0] - logz


def kernel(inputs, emb_table, W1, b1, W2, b2):
    idx = jnp.pad(inputs, (0, IDX_PAD - CTX))
    emb2 = emb_table.reshape(VOCAB_SIZE // 2, 2 * EMB_DIM)
    gathered = _sc_gather(emb2, idx // 2)
    e2d = gathered.reshape(1, IDX_PAD * 2 * EMB_DIM)[:, : CTX * 2 * EMB_DIM]

    par = (idx % 2)[:CTX]
    mask = (jnp.arange(2 * EMB_DIM)[None, :] // EMB_DIM == par[:, None])
    maskf = mask.astype(jnp.float32).reshape(1, CTX * 2 * EMB_DIM)

    w1r = W1.reshape(HID, CTX, EMB_DIM)
    w1dup = jnp.concatenate([w1r, w1r], axis=2).reshape(
        HID, CTX * 2 * EMB_DIM).astype(jnp.bfloat16)

    b1r = b1.reshape(1, HID)
    b2v = b2.reshape(N_TILES, 1, V_TILE)

    logits, m, s = pl.pallas_call(
        _pass_a_body,
        grid=(N_TILES,),
        in_specs=[
            pl.BlockSpec((1, CTX * 2 * EMB_DIM), lambda i: (0, 0)),
            pl.BlockSpec((1, CTX * 2 * EMB_DIM), lambda i: (0, 0)),
            pl.BlockSpec((HID, CTX * 2 * EMB_DIM), lambda i: (0, 0)),
            pl.BlockSpec((1, HID), lambda i: (0, 0)),
            pl.BlockSpec((V_TILE, HID), lambda i: (i, 0)),
            pl.BlockSpec((1, 1, V_TILE), lambda i: (i, 0, 0)),
        ],
        out_specs=[
            pl.BlockSpec((1, 1, V_TILE), lambda i: (i, 0, 0)),
            pl.BlockSpec((1, 1, 1), lambda i: (i, 0, 0)),
            pl.BlockSpec((1, 1, 1), lambda i: (i, 0, 0)),
        ],
        out_shape=[
            jax.ShapeDtypeStruct((N_TILES, 1, V_TILE), jnp.float32),
            jax.ShapeDtypeStruct((N_TILES, 1, 1), jnp.float32),
            jax.ShapeDtypeStruct((N_TILES, 1, 1), jnp.float32),
        ],
        scratch_shapes=[pltpu.VMEM((1, HID), jnp.bfloat16)],
    )(e2d, maskf, w1dup, b1r, W2, b2v)

    out = pl.pallas_call(
        _pass_b_body,
        grid=(N_TILES,),
        in_specs=[
            pl.BlockSpec((1, 1, V_TILE), lambda i: (i, 0, 0)),
            pl.BlockSpec((N_TILES, 1, 1), lambda i: (0, 0, 0)),
            pl.BlockSpec((N_TILES, 1, 1), lambda i: (0, 0, 0)),
        ],
        out_specs=pl.BlockSpec((1, 1, V_TILE), lambda i: (i, 0, 0)),
        out_shape=jax.ShapeDtypeStruct((N_TILES, 1, V_TILE), jnp.float32),
    )(logits, m, s)

    return out.reshape(1, VOCAB_SIZE)


# speedup vs baseline: 1.4905x; 1.4905x over previous
import jax
import jax.numpy as jnp
from jax.experimental import pallas as pl
from jax.experimental.pallas import tpu as pltpu

VOCAB_SIZE = 1000000
EMB_DIM = 64
CTX = 200
HID = 128

V_TILE = 20000
N_TILES = VOCAB_SIZE // V_TILE


def _gather_body(idx_ref, emb_ref, o_ref, sem):
    def issue(t, carry):
        r = idx_ref[t]
        pltpu.make_async_copy(
            emb_ref.at[pl.ds(r, 1), :], o_ref.at[pl.ds(t, 1), :], sem
        ).start()
        return carry

    jax.lax.fori_loop(0, CTX, issue, 0)

    def drain(t, carry):
        pltpu.make_async_copy(
            emb_ref.at[pl.ds(0, 1), :], o_ref.at[pl.ds(t, 1), :], sem
        ).wait()
        return carry

    jax.lax.fori_loop(0, CTX, drain, 0)


def _pass_a_body(e_ref, w1_ref, b1_ref, w2_ref, b2_ref,
                 out_ref, m_ref, s_ref, h_ref):
    i = pl.program_id(0)

    @pl.when(i == 0)
    def _():
        eb = e_ref[...].astype(jnp.bfloat16)
        hpre = jax.lax.dot_general(
            eb, w1_ref[...], (((1,), (1,)), ((), ())),
            preferred_element_type=jnp.float32)
        h = jnp.maximum(hpre + b1_ref[...], 0.0)
        h_ref[...] = h.astype(jnp.bfloat16)

    w2b = w2_ref[...].astype(jnp.bfloat16)
    logits = jax.lax.dot_general(
        h_ref[...], w2b, (((1,), (1,)), ((), ())),
        preferred_element_type=jnp.float32)
    row = logits + b2_ref[0]
    out_ref[0] = row
    m = jnp.max(row)
    m_ref[...] = jnp.full((1, 1, 1), m, jnp.float32)
    s_ref[...] = jnp.full((1, 1, 1), jnp.sum(jnp.exp(row - m)), jnp.float32)


def _pass_b_body(l_ref, m_ref, s_ref, o_ref):
    m_all = m_ref[...]
    big = jnp.max(m_all)
    total = jnp.sum(s_ref[...] * jnp.exp(m_all - big))
    logz = big + jnp.log(total)
    o_ref[0] = l_ref[0] - logz


def kernel(inputs, emb_table, W1, b1, W2, b2):
    rows = pl.pallas_call(
        _gather_body,
        in_specs=[
            pl.BlockSpec(memory_space=pltpu.SMEM),
            pl.BlockSpec(memory_space=pltpu.MemorySpace.HBM),
        ],
        out_specs=pl.BlockSpec(memory_space=pltpu.VMEM),
        out_shape=jax.ShapeDtypeStruct((CTX, EMB_DIM), jnp.float32),
        scratch_shapes=[pltpu.SemaphoreType.DMA],
    )(inputs, emb_table)
    e2d = rows.reshape(1, CTX * EMB_DIM)

    w1b = W1.astype(jnp.bfloat16)
    b1r = b1.reshape(1, HID)
    b2v = b2.reshape(N_TILES, 1, V_TILE)

    logits, m, s = pl.pallas_call(
        _pass_a_body,
        grid=(N_TILES,),
        in_specs=[
            pl.BlockSpec((1, CTX * EMB_DIM), lambda i: (0, 0)),
            pl.BlockSpec((HID, CTX * EMB_DIM), lambda i: (0, 0)),
            pl.BlockSpec((1, HID), lambda i: (0, 0)),
            pl.BlockSpec((V_TILE, HID), lambda i: (i, 0)),
            pl.BlockSpec((1, 1, V_TILE), lambda i: (i, 0, 0)),
        ],
        out_specs=[
            pl.BlockSpec((1, 1, V_TILE), lambda i: (i, 0, 0)),
            pl.BlockSpec((1, 1, 1), lambda i: (i, 0, 0)),
            pl.BlockSpec((1, 1, 1), lambda i: (i, 0, 0)),
        ],
        out_shape=[
            jax.ShapeDtypeStruct((N_TILES, 1, V_TILE), jnp.float32),
            jax.ShapeDtypeStruct((N_TILES, 1, 1), jnp.float32),
            jax.ShapeDtypeStruct((N_TILES, 1, 1), jnp.float32),
        ],
        scratch_shapes=[pltpu.VMEM((1, HID), jnp.bfloat16)],
    )(e2d, w1b, b1r, W2, b2v)

    out = pl.pallas_call(
        _pass_b_body,
        grid=(N_TILES,),
        in_specs=[
            pl.BlockSpec((1, 1, V_TILE), lambda i: (i, 0, 0)),
            pl.BlockSpec((N_TILES, 1, 1), lambda i: (0, 0, 0)),
            pl.BlockSpec((N_TILES, 1, 1), lambda i: (0, 0, 0)),
        ],
        out_specs=pl.BlockSpec((1, 1, V_TILE), lambda i: (i, 0, 0)),
        out_shape=jax.ShapeDtypeStruct((N_TILES, 1, V_TILE), jnp.float32),
    )(logits, m, s)

    return out.reshape(1, VOCAB_SIZE)


# speedup vs baseline: 5.0061x; 3.3586x over previous
import jax
import jax.numpy as jnp
from jax.experimental import pallas as pl
from jax.experimental.pallas import tpu as pltpu

VOCAB_SIZE = 1000000
EMB_DIM = 64
CTX = 200
HID = 128
FLAT = CTX * EMB_DIM

V_TILE = 20480
N_TILES = -(-VOCAB_SIZE // V_TILE)


def _prep_body(idx_ref, embt_ref, oh_ref, w1_ref, b1_ref, h_ref, e2_ref, sem):
    def issue(t, carry):
        r = idx_ref[t]
        start = pl.multiple_of((r // 128) * 128, 128)
        pltpu.make_async_copy(
            embt_ref.at[:, pl.ds(start, 128)],
            e2_ref.at[pl.ds(t * EMB_DIM, EMB_DIM), :],
            sem,
        ).start()
        return carry

    jax.lax.fori_loop(0, CTX, issue, 0)

    def drain(t, carry):
        pltpu.make_async_copy(
            embt_ref.at[:, pl.ds(0, 128)],
            e2_ref.at[pl.ds(t * EMB_DIM, EMB_DIM), :],
            sem,
        ).wait()
        return carry

    jax.lax.fori_loop(0, CTX, drain, 0)

    esel = jnp.sum(e2_ref[...] * oh_ref[...], axis=1, keepdims=True)
    hpre = jax.lax.dot_general(
        w1_ref[...], esel.astype(jnp.bfloat16), (((1,), (0,)), ((), ())),
        preferred_element_type=jnp.float32)
    h = jnp.maximum(hpre + b1_ref[...], 0.0)
    h_ref[...] = jnp.transpose(h, (1, 0))


def _pass_a_body(h_ref, w2_ref, b2_ref, out_ref, m_ref, s_ref):
    i = pl.program_id(0)
    w2b = w2_ref[...].astype(jnp.bfloat16)
    logits = jax.lax.dot_general(
        h_ref[...].astype(jnp.bfloat16), w2b, (((1,), (1,)), ((), ())),
        preferred_element_type=jnp.float32)
    row = logits + b2_ref[...][None, :]
    lane = jax.lax.broadcasted_iota(jnp.int32, (1, V_TILE), 1)
    row = jnp.where(lane < VOCAB_SIZE - i * V_TILE, row, -1e30)
    out_ref[...] = row
    m = jnp.max(row)
    m_ref[...] = jnp.full((1, 1, 1), m, jnp.float32)
    s_ref[...] = jnp.full((1, 1, 1), jnp.sum(jnp.exp(row - m)), jnp.float32)


def _pass_b_body(l_ref, m_ref, s_ref, o_ref):
    m_all = m_ref[...]
    big = jnp.max(m_all)
    total = jnp.sum(s_ref[...] * jnp.exp(m_all - big))
    logz = big + jnp.log(total)
    o_ref[...] = l_ref[...] - logz


def kernel(inputs, emb_table, W1, b1, W2, b2):
    oh = jax.nn.one_hot(inputs % 128, 128, dtype=jnp.bfloat16)
    oh_flat = jnp.repeat(oh, EMB_DIM, axis=0)
    w1b = W1.astype(jnp.bfloat16)
    b1c = b1.reshape(HID, 1)

    h = pl.pallas_call(
        _prep_body,
        in_specs=[
            pl.BlockSpec(memory_space=pltpu.SMEM),
            pl.BlockSpec(memory_space=pltpu.MemorySpace.HBM),
            pl.BlockSpec(memory_space=pltpu.VMEM),
            pl.BlockSpec(memory_space=pltpu.VMEM),
            pl.BlockSpec(memory_space=pltpu.VMEM),
        ],
        out_specs=pl.BlockSpec(memory_space=pltpu.VMEM),
        out_shape=jax.ShapeDtypeStruct((1, HID), jnp.float32),
        scratch_shapes=[
            pltpu.VMEM((FLAT, 128), jnp.float32),
            pltpu.SemaphoreType.DMA,
        ],
    )(inputs, emb_table.T, oh_flat, w1b, b1c)

    logits, m, s = pl.pallas_call(
        _pass_a_body,
        grid=(N_TILES,),
        in_specs=[
            pl.BlockSpec((1, HID), lambda i: (0, 0)),
            pl.BlockSpec((V_TILE, HID), lambda i: (i, 0)),
            pl.BlockSpec((V_TILE,), lambda i: (i,)),
        ],
        out_specs=[
            pl.BlockSpec((1, V_TILE), lambda i: (0, i)),
            pl.BlockSpec((1, 1, 1), lambda i: (i, 0, 0)),
            pl.BlockSpec((1, 1, 1), lambda i: (i, 0, 0)),
        ],
        out_shape=[
            jax.ShapeDtypeStruct((1, VOCAB_SIZE), jnp.float32),
            jax.ShapeDtypeStruct((N_TILES, 1, 1), jnp.float32),
            jax.ShapeDtypeStruct((N_TILES, 1, 1), jnp.float32),
        ],
    )(h, W2, b2)

    out = pl.pallas_call(
        _pass_b_body,
        in_specs=[
            pl.BlockSpec(memory_space=pltpu.VMEM),
            pl.BlockSpec(memory_space=pltpu.VMEM),
            pl.BlockSpec(memory_space=pltpu.VMEM),
        ],
        out_specs=pl.BlockSpec(memory_space=pltpu.VMEM),
        out_shape=jax.ShapeDtypeStruct((1, VOCAB_SIZE), jnp.float32),
    )(logits, m, s)

    return out


# speedup vs baseline: 5.1724x; 1.0332x over previous
import jax
import jax.numpy as jnp
from jax.experimental import pallas as pl
from jax.experimental.pallas import tpu as pltpu

VOCAB_SIZE = 1000000
EMB_DIM = 64
CTX = 200
HID = 128
FLAT = CTX * EMB_DIM

V_TILE = 30720
N_TILES = -(-VOCAB_SIZE // V_TILE)


def _pass_a_body(idx_ref, embt_ref, oh_ref, w1_ref, b1_ref, w2_ref, b2_ref,
                 out_ref, m_ref, s_ref, h_ref, e2_ref, mrun_ref, srun_ref, sem):
    i = pl.program_id(0)

    @pl.when(i == 0)
    def _():
        def issue(t, carry):
            r = idx_ref[t]
            start = pl.multiple_of((r // 128) * 128, 128)
            pltpu.make_async_copy(
                embt_ref.at[:, pl.ds(start, 128)],
                e2_ref.at[pl.ds(t * EMB_DIM, EMB_DIM), :],
                sem,
            ).start()
            return carry

        jax.lax.fori_loop(0, CTX, issue, 0)

        def drain(t, carry):
            pltpu.make_async_copy(
                embt_ref.at[:, pl.ds(0, 128)],
                e2_ref.at[pl.ds(t * EMB_DIM, EMB_DIM), :],
                sem,
            ).wait()
            return carry

        jax.lax.fori_loop(0, CTX, drain, 0)

        esel = jnp.sum(e2_ref[...] * oh_ref[...], axis=1, keepdims=True)
        hpre = jax.lax.dot_general(
            w1_ref[...].astype(jnp.bfloat16), esel.astype(jnp.bfloat16),
            (((1,), (0,)), ((), ())), preferred_element_type=jnp.float32)
        h = jnp.maximum(jnp.transpose(hpre, (1, 0)) + b1_ref[...], 0.0)
        h_ref[...] = h.astype(jnp.bfloat16)
        mrun_ref[...] = jnp.full((1, HID), -1e30, jnp.float32)
        srun_ref[...] = jnp.zeros((1, HID), jnp.float32)

    w2b = w2_ref[...].astype(jnp.bfloat16)
    logits = jax.lax.dot_general(
        h_ref[...], w2b, (((1,), (1,)), ((), ())),
        preferred_element_type=jnp.float32)
    row = logits + b2_ref[...][None, :]
    lane = jax.lax.broadcasted_iota(jnp.int32, (1, V_TILE), 1)
    row = jnp.where(lane < VOCAB_SIZE - i * V_TILE, row, -1e30)
    out_ref[...] = row
    mt = jnp.max(row)
    st = jnp.sum(jnp.exp(row - mt))
    mnew = jnp.maximum(mrun_ref[...], mt)
    srun_ref[...] = (srun_ref[...] * jnp.exp(mrun_ref[...] - mnew)
                     + st * jnp.exp(mt - mnew))
    mrun_ref[...] = mnew

    @pl.when(i == N_TILES - 1)
    def _():
        m_ref[...] = mrun_ref[...]
        s_ref[...] = srun_ref[...]


def _pass_b_body(l_ref, m_ref, s_ref, o_ref):
    logz = jnp.max(m_ref[...]) + jnp.log(jnp.max(s_ref[...]))
    o_ref[...] = l_ref[...] - logz


def kernel(inputs, emb_table, W1, b1, W2, b2):
    oh = jax.nn.one_hot(inputs % 128, 128, dtype=jnp.bfloat16)
    oh_flat = jnp.repeat(oh, EMB_DIM, axis=0)
    b1r = b1.reshape(1, HID)

    logits, m, s = pl.pallas_call(
        _pass_a_body,
        grid=(N_TILES,),
        in_specs=[
            pl.BlockSpec(memory_space=pltpu.SMEM),
            pl.BlockSpec(memory_space=pltpu.MemorySpace.HBM),
            pl.BlockSpec((FLAT, 128), lambda i: (0, 0)),
            pl.BlockSpec((HID, FLAT), lambda i: (0, 0)),
            pl.BlockSpec((1, HID), lambda i: (0, 0)),
            pl.BlockSpec((V_TILE, HID), lambda i: (i, 0)),
            pl.BlockSpec((V_TILE,), lambda i: (i,)),
        ],
        out_specs=[
            pl.BlockSpec((1, V_TILE), lambda i: (0, i)),
            pl.BlockSpec((1, HID), lambda i: (0, 0)),
            pl.BlockSpec((1, HID), lambda i: (0, 0)),
        ],
        out_shape=[
            jax.ShapeDtypeStruct((1, VOCAB_SIZE), jnp.float32),
            jax.ShapeDtypeStruct((1, HID), jnp.float32),
            jax.ShapeDtypeStruct((1, HID), jnp.float32),
        ],
        scratch_shapes=[
            pltpu.VMEM((1, HID), jnp.bfloat16),
            pltpu.VMEM((FLAT, 128), jnp.float32),
            pltpu.VMEM((1, HID), jnp.float32),
            pltpu.VMEM((1, HID), jnp.float32),
            pltpu.SemaphoreType.DMA,
        ],
    )(inputs, emb_table.T, oh_flat, W1, b1r, W2, b2)

    out = pl.pallas_call(
        _pass_b_body,
        in_specs=[
            pl.BlockSpec(memory_space=pltpu.VMEM),
            pl.BlockSpec(memory_space=pltpu.VMEM),
            pl.BlockSpec(memory_space=pltpu.VMEM),
        ],
        out_specs=pl.BlockSpec(memory_space=pltpu.VMEM),
        out_shape=jax.ShapeDtypeStruct((1, VOCAB_SIZE), jnp.float32),
    )(logits, m, s)

    return out


# speedup vs baseline: 5.1960x; 1.0046x over previous
import jax
import jax.numpy as jnp
from jax.experimental import pallas as pl
from jax.experimental.pallas import tpu as pltpu

VOCAB_SIZE = 1000000
EMB_DIM = 64
CTX = 200
HID = 128
FLAT = CTX * EMB_DIM

V_TILE = 20480
N_TILES = -(-VOCAB_SIZE // V_TILE)


def _pass_a_body(idx_ref, embt_ref, oh_ref, w1_ref, b1_ref, w2_ref, b2_ref,
                 out_ref, m_ref, s_ref, h_ref, e2_ref, mrun_ref, srun_ref, sem):
    i = pl.program_id(0)

    @pl.when(i == 0)
    def _():
        def issue(t, carry):
            r = idx_ref[t]
            start = pl.multiple_of((r // 128) * 128, 128)
            pltpu.make_async_copy(
                embt_ref.at[:, pl.ds(start, 128)],
                e2_ref.at[pl.ds(t * EMB_DIM, EMB_DIM), :],
                sem,
            ).start()
            return carry

        jax.lax.fori_loop(0, CTX, issue, 0)

        def drain(t, carry):
            pltpu.make_async_copy(
                embt_ref.at[:, pl.ds(0, 128)],
                e2_ref.at[pl.ds(t * EMB_DIM, EMB_DIM), :],
                sem,
            ).wait()
            return carry

        jax.lax.fori_loop(0, CTX, drain, 0)

        esel = jnp.sum(e2_ref[...] * oh_ref[...], axis=1, keepdims=True)
        hpre = jax.lax.dot_general(
            w1_ref[...].astype(jnp.bfloat16), esel.astype(jnp.bfloat16),
            (((1,), (0,)), ((), ())), preferred_element_type=jnp.float32)
        h = jnp.maximum(jnp.transpose(hpre, (1, 0)) + b1_ref[...], 0.0)
        h_ref[...] = h.astype(jnp.bfloat16)
        mrun_ref[...] = jnp.full((1, HID), -1e30, jnp.float32)
        srun_ref[...] = jnp.zeros((1, HID), jnp.float32)

    w2b = w2_ref[...].astype(jnp.bfloat16)
    logits = jax.lax.dot_general(
        h_ref[...], w2b, (((1,), (1,)), ((), ())),
        preferred_element_type=jnp.float32)
    row = logits + b2_ref[...][None, :]
    lane = jax.lax.broadcasted_iota(jnp.int32, (1, V_TILE), 1)
    row = jnp.where(lane < VOCAB_SIZE - i * V_TILE, row, -1e30)
    out_ref[...] = row
    mt = jnp.max(row)
    st = jnp.sum(jnp.exp(row - mt))
    mnew = jnp.maximum(mrun_ref[...], mt)
    srun_ref[...] = (srun_ref[...] * jnp.exp(mrun_ref[...] - mnew)
                     + st * jnp.exp(mt - mnew))
    mrun_ref[...] = mnew

    @pl.when(i == N_TILES - 1)
    def _():
        m_ref[...] = mrun_ref[...]
        s_ref[...] = srun_ref[...]


def _pass_b_body(l_ref, m_ref, s_ref, o_ref):
    logz = jnp.max(m_ref[...]) + jnp.log(jnp.max(s_ref[...]))
    o_ref[...] = l_ref[...] - logz


def kernel(inputs, emb_table, W1, b1, W2, b2):
    oh = jax.nn.one_hot(inputs % 128, 128, dtype=jnp.bfloat16)
    oh_flat = jnp.repeat(oh, EMB_DIM, axis=0)
    b1r = b1.reshape(1, HID)

    logits, m, s = pl.pallas_call(
        _pass_a_body,
        grid=(N_TILES,),
        in_specs=[
            pl.BlockSpec(memory_space=pltpu.SMEM),
            pl.BlockSpec(memory_space=pltpu.MemorySpace.HBM),
            pl.BlockSpec((FLAT, 128), lambda i: (0, 0)),
            pl.BlockSpec((HID, FLAT), lambda i: (0, 0)),
            pl.BlockSpec((1, HID), lambda i: (0, 0)),
            pl.BlockSpec((V_TILE, HID), lambda i: (i, 0)),
            pl.BlockSpec((V_TILE,), lambda i: (i,)),
        ],
        out_specs=[
            pl.BlockSpec((1, V_TILE), lambda i: (0, i)),
            pl.BlockSpec((1, HID), lambda i: (0, 0)),
            pl.BlockSpec((1, HID), lambda i: (0, 0)),
        ],
        out_shape=[
            jax.ShapeDtypeStruct((1, VOCAB_SIZE), jnp.float32),
            jax.ShapeDtypeStruct((1, HID), jnp.float32),
            jax.ShapeDtypeStruct((1, HID), jnp.float32),
        ],
        scratch_shapes=[
            pltpu.VMEM((1, HID), jnp.bfloat16),
            pltpu.VMEM((FLAT, 128), jnp.float32),
            pltpu.VMEM((1, HID), jnp.float32),
            pltpu.VMEM((1, HID), jnp.float32),
            pltpu.SemaphoreType.DMA,
        ],
    )(inputs, emb_table.T, oh_flat, W1, b1r, W2, b2)

    out = pl.pallas_call(
        _pass_b_body,
        in_specs=[
            pl.BlockSpec(memory_space=pltpu.VMEM),
            pl.BlockSpec(memory_space=pltpu.VMEM),
            pl.BlockSpec(memory_space=pltpu.VMEM),
        ],
        out_specs=pl.BlockSpec(memory_space=pltpu.VMEM),
        out_shape=jax.ShapeDtypeStruct((1, VOCAB_SIZE), jnp.float32),
    )(logits, m, s)

    return out
